# baseline (speedup 1.0000x reference)
import functools

import jax
import jax.numpy as jnp
from jax import lax
from jax.experimental import pallas as pl
from jax.experimental.pallas import tpu as pltpu
from jax.experimental.pallas import tpu_sc as plsc

_B = 16384
_F = 26
_V = 100000
_D = 16

_NC = 2
_NS = 16
_NW = _NC * _NS
_NT = _F * _D
_TPW = _NT // _NW
_PH = 4096
_NPH = _B // _PH


def _lookup_body(idx_hbm, tab_hbm, out_hbm, tabv, idxv, outv0, outv1, wsem, tsem):
    wid = lax.axis_index("s") * _NC + lax.axis_index("c")
    outv = (outv0, outv1)
    lax.fori_loop(0, lax.axis_index("s") * 150, lambda i, c: c + 1, 0)

    def task(k, carry):
        t = wid * _TPW + k
        f = t // _D
        d = t % _D
        cp = pltpu.make_async_copy(tab_hbm.at[f, d, :], tabv, tsem)
        cp.start()

        @pl.when((k == 0) | (d == 0))
        def _():
            pltpu.sync_copy(idx_hbm.at[f, :], idxv)

        cp.wait()
        for p in range(_NPH):
            b = p % 2
            ov = outv[b]
            p0 = p * _PH

            @pl.when((k > 0) | (p >= 2))
            def _():
                pltpu.make_async_copy(
                    ov, out_hbm.at[t, pl.ds(p0, _PH)], wsem
                ).wait()

            @plsc.parallel_loop(0, _PH, 16, unroll=8)
            def gath(i):
                ov[pl.ds(i, 16)] = plsc.load_gather(
                    tabv, [idxv[pl.ds(p0 + i, 16)]]
                )

            pltpu.async_copy(ov, out_hbm.at[t, pl.ds(p0, _PH)], wsem)
        return carry

    lax.fori_loop(0, _TPW, task, 0)
    t_last = wid * _TPW + (_TPW - 1)
    for p in range(_NPH - 2, _NPH):
        pltpu.make_async_copy(
            outv[p % 2], out_hbm.at[t_last, pl.ds(p * _PH, _PH)], wsem
        ).wait()


_lookup = functools.partial(
    pl.kernel,
    mesh=plsc.VectorSubcoreMesh(core_axis_name="c", subcore_axis_name="s"),
    out_type=jax.ShapeDtypeStruct((_NT, _B), jnp.float32),
    scratch_types=[
        pltpu.VMEM((_V,), jnp.float32),
        pltpu.VMEM((_B,), jnp.int32),
        pltpu.VMEM((_PH,), jnp.float32),
        pltpu.VMEM((_PH,), jnp.float32),
        pltpu.SemaphoreType.DMA,
        pltpu.SemaphoreType.DMA,
    ],
    compiler_params=pltpu.CompilerParams(
        use_tc_tiling_on_sc=True, needs_layout_passes=False
    ),
)(_lookup_body)


@jax.jit
def kernel(indices, tables):
    idx_t = indices.T
    tab_t = tables.transpose(0, 2, 1)
    out_t = _lookup(idx_t, tab_t)
    return out_t.T.reshape(_B, _F * _D)


# speedup vs baseline: 1.0009x; 1.0009x over previous
import functools

import jax
import jax.numpy as jnp
from jax import lax
from jax.experimental import pallas as pl
from jax.experimental.pallas import tpu as pltpu
from jax.experimental.pallas import tpu_sc as plsc

_B = 16384
_F = 26
_V = 100000
_D = 16

_NC = 2
_NS = 16
_NW = _NC * _NS
_NT = _F * _D
_TPW = _NT // _NW
_PH = 4096
_NPH = _B // _PH


def _lookup_body(idx_hbm, tab_hbm, out_hbm, tabv, idxv, outv0, outv1, wsem, tsem):
    wid = lax.axis_index("s") * _NC + lax.axis_index("c")
    outv = (outv0, outv1)

    def task(k, carry):
        t = wid * _TPW + k
        f = t // _D
        d = t % _D
        cp = pltpu.make_async_copy(tab_hbm.at[f, d, :], tabv, tsem)
        cp.start()

        @pl.when((k == 0) | (d == 0))
        def _():
            pltpu.sync_copy(idx_hbm.at[f, :], idxv)

        cp.wait()
        for p in range(_NPH):
            b = p % 2
            ov = outv[b]
            p0 = p * _PH

            @pl.when((k > 0) | (p >= 2))
            def _():
                pltpu.make_async_copy(
                    ov, out_hbm.at[t, pl.ds(p0, _PH)], wsem
                ).wait()

            @plsc.parallel_loop(0, _PH, 16, unroll=8)
            def gath(i):
                ov[pl.ds(i, 16)] = plsc.load_gather(
                    tabv, [idxv[pl.ds(p0 + i, 16)]]
                )

            pltpu.async_copy(ov, out_hbm.at[t, pl.ds(p0, _PH)], wsem)
        return carry

    lax.fori_loop(0, _TPW, task, 0)
    t_last = wid * _TPW + (_TPW - 1)
    for p in range(_NPH - 2, _NPH):
        pltpu.make_async_copy(
            outv[p % 2], out_hbm.at[t_last, pl.ds(p * _PH, _PH)], wsem
        ).wait()


_lookup = functools.partial(
    pl.kernel,
    mesh=plsc.VectorSubcoreMesh(core_axis_name="c", subcore_axis_name="s"),
    out_type=jax.ShapeDtypeStruct((_NT, _B), jnp.float32),
    scratch_types=[
        pltpu.VMEM((_V,), jnp.float32),
        pltpu.VMEM((_B,), jnp.int32),
        pltpu.VMEM((_PH,), jnp.float32),
        pltpu.VMEM((_PH,), jnp.float32),
        pltpu.SemaphoreType.DMA,
        pltpu.SemaphoreType.DMA,
    ],
    compiler_params=pltpu.CompilerParams(
        use_tc_tiling_on_sc=True, needs_layout_passes=False
    ),
)(_lookup_body)


@jax.jit
def kernel(indices, tables):
    idx_t = indices.T
    tab_t = tables.transpose(0, 2, 1)
    out_t = _lookup(idx_t, tab_t)
    return out_t.T.reshape(_B, _F * _D)
